# Initial kernel scaffold; baseline (speedup 1.0000x reference)
import jax
import jax.numpy as jnp
from jax.experimental import pallas as pl


def kernel(entity_embed, edge_index, edge_att, W, b):
    raise NotImplementedError("write your pallas kernel here")


# speedup vs baseline: 4.2548x; 4.2548x over previous
import functools

import jax
import jax.numpy as jnp
from jax import lax
from jax.experimental import pallas as pl
from jax.experimental.pallas import tpu as pltpu
from jax.experimental.pallas import tpu_sc as plsc

N_NODES = 10000
N_EDGES = 320000
D = 128
NC, NS, L = 2, 16, 16
NW = NC * NS
K = 128
NCHUNK = N_EDGES // K
ACC_ROWS = 10240
VR = D // L


def _sc_body(src_h, dst_h, att_h, ent_h, out_h,
             acc, src_buf, dst_buf, att_buf, gbuf, gsem):
    c = lax.axis_index("c")
    s = lax.axis_index("s")
    w = s * NC + c

    def zero_row(r, _):
        for k in range(VR):
            gbuf[r, pl.ds(k * L, L)] = jnp.zeros((L,), jnp.float32)
        return _
    lax.fori_loop(0, K, zero_row, 0)
    for j in range(ACC_ROWS // K // NS):
        pltpu.sync_copy(gbuf, acc.at[pl.ds((s * 5 + j) * K, K)])
    plsc.subcore_barrier()

    ntrips = jnp.where(w < NCHUNK - (NCHUNK // NW) * NW,
                       NCHUNK // NW + 1, NCHUNK // NW)

    def chunk_body(j, _):
        base = (w + NW * j) * K
        pltpu.sync_copy(src_h.at[pl.ds(base, K)], src_buf)
        pltpu.sync_copy(dst_h.at[pl.ds(base, K)], dst_buf)
        pltpu.sync_copy(att_h.at[pl.ds(base, K)], att_buf)
        pltpu.async_copy(ent_h.at[src_buf], gbuf, gsem).wait()

        def edge_group(g, _):
            av_vec = att_buf[pl.ds(g * L, L)]
            for i in range(L):
                av = av_vec[i]
                e = g * L + i
                for k in range(VR):
                    gbuf[e, pl.ds(k * L, L)] = gbuf[e, pl.ds(k * L, L)] * av
            return _
        lax.fori_loop(0, K // L, edge_group, 0)
        pltpu.sync_copy(gbuf, acc.at[dst_buf], add=True)
        return _
    lax.fori_loop(0, ntrips, chunk_body, 0)
    plsc.subcore_barrier()

    rpt = ACC_ROWS // NS
    pltpu.sync_copy(acc.at[pl.ds(s * rpt, rpt)],
                    out_h.at[c, pl.ds(s * rpt, rpt)])


_sc_call = functools.partial(
    pl.kernel,
    out_type=jax.ShapeDtypeStruct((NC, ACC_ROWS, D), jnp.float32),
    mesh=plsc.VectorSubcoreMesh(core_axis_name="c", subcore_axis_name="s",
                                num_cores=NC, num_subcores=NS),
    scratch_types=[
        pltpu.VMEM_SHARED((ACC_ROWS, D), jnp.float32),
        pltpu.VMEM((K,), jnp.int32),
        pltpu.VMEM((K,), jnp.int32),
        pltpu.VMEM((K,), jnp.float32),
        pltpu.VMEM((K, D), jnp.float32),
        pltpu.SemaphoreType.DMA,
    ],
)(_sc_body)


def _tc_body(ent_ref, p0_ref, p1_ref, w_ref, b_ref, out_ref):
    x = ent_ref[...] + p0_ref[0] + p1_ref[0]
    y = lax.dot_general(x, w_ref[...], (((1,), (1,)), ((), ())),
                        preferred_element_type=jnp.float32) + b_ref[...]
    out_ref[...] = jnp.where(y >= 0, y, 0.01 * y)


_TC_BLK = 80

_tc_call = pl.pallas_call(
    _tc_body,
    grid=(N_NODES // _TC_BLK,),
    in_specs=[
        pl.BlockSpec((_TC_BLK, D), lambda i: (i, 0)),
        pl.BlockSpec((1, _TC_BLK, D), lambda i: (0, i, 0)),
        pl.BlockSpec((1, _TC_BLK, D), lambda i: (1, i, 0)),
        pl.BlockSpec((D, D), lambda i: (0, 0)),
        pl.BlockSpec((1, D), lambda i: (0, 0)),
    ],
    out_specs=pl.BlockSpec((_TC_BLK, D), lambda i: (i, 0)),
    out_shape=jax.ShapeDtypeStruct((N_NODES, D), jnp.float32),
)


def kernel(entity_embed, edge_index, edge_att, W, b):
    src = edge_index[0]
    dst = edge_index[1]
    att = edge_att.reshape(-1)
    partial = _sc_call(src, dst, att, entity_embed)
    return _tc_call(entity_embed, partial, partial, W, b.reshape(1, D))
